# initial kernel scaffold (unmeasured)
import jax
import jax.numpy as jnp
from jax import lax
from jax.experimental import pallas as pl
from jax.experimental.pallas import tpu as pltpu


def kernel(
    x,
):
    def body(*refs):
        pass

    out_shape = jax.ShapeDtypeStruct(..., jnp.float32)
    return pl.pallas_call(body, out_shape=out_shape)(...)


# baseline (device time: 594210 ns/iter reference)
import jax
import jax.numpy as jnp
from jax import lax
from jax.experimental import pallas as pl
from jax.experimental.pallas import tpu as pltpu

N_DEV = 4


def kernel(x):
    m, n = x.shape
    chunk = m // N_DEV
    n_steps = N_DEV - 1
    n_sems = 2 * n_steps

    def body(x_ref, out_ref, comm_ref, x_stage, send_sems, recv_sems, copy_sem):
        my_x = lax.axis_index("x")
        my_y = lax.axis_index("y")
        my_z = lax.axis_index("z")
        nxt = (my_z + 1) % N_DEV
        prv = (my_z - 1) % N_DEV
        nxt_id = (my_x, my_y, nxt)

        barrier_sem = pltpu.get_barrier_semaphore()
        for nbr in (nxt, prv):
            pl.semaphore_signal(
                barrier_sem, inc=1,
                device_id=(my_x, my_y, nbr),
                device_id_type=pl.DeviceIdType.MESH,
            )
        pl.semaphore_wait(barrier_sem, 2)

        for s in range(n_steps):
            cs = (my_z - s) % N_DEV
            cr = (my_z - s - 1) % N_DEV
            src = (
                x_ref.at[pl.ds(cs * chunk, chunk), :]
                if s == 0
                else comm_ref.at[s - 1]
            )
            rdma = pltpu.make_async_remote_copy(
                src_ref=src,
                dst_ref=comm_ref.at[s],
                send_sem=send_sems.at[s],
                recv_sem=recv_sems.at[s],
                device_id=nxt_id,
                device_id_type=pl.DeviceIdType.MESH,
            )
            rdma.start()
            cp = pltpu.make_async_copy(
                x_ref.at[pl.ds(cr * chunk, chunk), :], x_stage, copy_sem
            )
            cp.start()
            cp.wait()
            rdma.wait()
            comm_ref[s] = comm_ref[s] + x_stage[...]

        cstar = (my_z + 1) % N_DEV
        cp = pltpu.make_async_copy(
            comm_ref.at[n_steps - 1],
            out_ref.at[pl.ds(cstar * chunk, chunk), :],
            copy_sem,
        )
        cp.start()
        cp.wait()

        for s in range(n_steps):
            send_slot = n_steps - 1 if s == 0 else s - 1
            recv_slot = s
            cg = (my_z - s) % N_DEV
            rdma = pltpu.make_async_remote_copy(
                src_ref=comm_ref.at[send_slot],
                dst_ref=comm_ref.at[recv_slot],
                send_sem=send_sems.at[n_steps + s],
                recv_sem=recv_sems.at[n_steps + s],
                device_id=nxt_id,
                device_id_type=pl.DeviceIdType.MESH,
            )
            rdma.start()
            rdma.wait()
            cp = pltpu.make_async_copy(
                comm_ref.at[recv_slot],
                out_ref.at[pl.ds(cg * chunk, chunk), :],
                copy_sem,
            )
            cp.start()
            cp.wait()

    return pl.pallas_call(
        body,
        out_shape=jax.ShapeDtypeStruct((m, n), x.dtype),
        in_specs=[pl.BlockSpec(memory_space=pl.ANY)],
        out_specs=pl.BlockSpec(memory_space=pl.ANY),
        scratch_shapes=[
            pltpu.VMEM((n_steps, chunk, n), x.dtype),
            pltpu.VMEM((chunk, n), x.dtype),
            pltpu.SemaphoreType.DMA((n_sems,)),
            pltpu.SemaphoreType.DMA((n_sems,)),
            pltpu.SemaphoreType.DMA,
        ],
        compiler_params=pltpu.CompilerParams(collective_id=0),
    )(x)


# device time: 297950 ns/iter; 1.9943x vs baseline; 1.9943x over previous
import jax
import jax.numpy as jnp
from jax import lax
from jax.experimental import pallas as pl
from jax.experimental.pallas import tpu as pltpu

N_X, N_Y, N_Z = 2, 4, 4
MESH = pl.DeviceIdType.MESH


def kernel(x):
    m, n = x.shape
    half = m // N_X
    quarter = half // N_Y
    piece = quarter // N_Z

    def body(
        x_ref, out_ref,
        comm, x_stage, quarter_buf, yblk, xblk,
        zs_send, zs_recv, za_send, za_recv,
        y_send, y_recv, x_send, x_recv,
        copy_sem, out_sems, xout_sems,
    ):
        my_x = lax.axis_index("x")
        my_y = lax.axis_index("y")
        my_z = lax.axis_index("z")
        z_nxt = (my_z + 1) % N_Z
        z_prv = (my_z - 1) % N_Z
        y_nxt = (my_y + 1) % N_Y
        y_prv = (my_y - 1) % N_Y
        x_peer = 1 - my_x
        base_q = my_x * half + my_y * quarter

        barrier_sem = pltpu.get_barrier_semaphore()
        for dev in (
            (my_x, my_y, z_nxt),
            (my_x, my_y, z_prv),
            (my_x, y_nxt, my_z),
            (my_x, y_prv, my_z),
            (x_peer, my_y, my_z),
        ):
            pl.semaphore_signal(
                barrier_sem, inc=1, device_id=dev, device_id_type=MESH
            )
        pl.semaphore_wait(barrier_sem, 5)

        for s in range(N_Z - 1):
            cs = (my_z - s) % N_Z
            cr = (my_z - s - 1) % N_Z
            src = (
                x_ref.at[pl.ds(base_q + cs * piece, piece), :]
                if s == 0
                else comm.at[s - 1]
            )
            rdma = pltpu.make_async_remote_copy(
                src_ref=src,
                dst_ref=comm.at[s],
                send_sem=zs_send.at[s],
                recv_sem=zs_recv.at[s],
                device_id=(my_x, my_y, z_nxt),
                device_id_type=MESH,
            )
            rdma.start()
            cp = pltpu.make_async_copy(
                x_ref.at[pl.ds(base_q + cr * piece, piece), :],
                x_stage,
                copy_sem,
            )
            cp.start()
            cp.wait()
            rdma.wait()
            comm[s] = comm[s] + x_stage[...]

        pstar = (my_z + 1) % N_Z
        cp = pltpu.make_async_copy(
            comm.at[N_Z - 2],
            quarter_buf.at[pl.ds(pstar * piece, piece), :],
            copy_sem,
        )
        cp.start()
        cp.wait()

        for s in range(N_Z - 1):
            pa = (my_z + 1 - s) % N_Z
            src = (
                comm.at[N_Z - 2]
                if s == 0
                else quarter_buf.at[pl.ds(pa * piece, piece), :]
            )
            rdma = pltpu.make_async_remote_copy(
                src_ref=src,
                dst_ref=quarter_buf.at[pl.ds(pa * piece, piece), :],
                send_sem=za_send.at[s],
                recv_sem=za_recv.at[s],
                device_id=(my_x, my_y, z_nxt),
                device_id_type=MESH,
            )
            rdma.start()
            rdma.wait()

        pending_copies = []
        cp = pltpu.make_async_copy(
            quarter_buf, out_ref.at[pl.ds(base_q, quarter), :], out_sems.at[0]
        )
        cp.start()
        pending_copies.append(cp)
        x_rdmas = []
        xr = pltpu.make_async_remote_copy(
            src_ref=quarter_buf,
            dst_ref=xblk.at[0],
            send_sem=x_send.at[0],
            recv_sem=x_recv.at[0],
            device_id=(x_peer, my_y, my_z),
            device_id_type=MESH,
        )
        xr.start()
        x_rdmas.append(xr)

        for t in range(N_Y - 1):
            g_r = (my_y - t - 1) % N_Y
            src = quarter_buf if t == 0 else yblk.at[t - 1]
            rdma = pltpu.make_async_remote_copy(
                src_ref=src,
                dst_ref=yblk.at[t],
                send_sem=y_send.at[t],
                recv_sem=y_recv.at[t],
                device_id=(my_x, y_nxt, my_z),
                device_id_type=MESH,
            )
            rdma.start()
            rdma.wait()
            rows = my_x * half + g_r * quarter
            cp = pltpu.make_async_copy(
                yblk.at[t], out_ref.at[pl.ds(rows, quarter), :],
                out_sems.at[1 + t],
            )
            cp.start()
            pending_copies.append(cp)
            xr = pltpu.make_async_remote_copy(
                src_ref=yblk.at[t],
                dst_ref=xblk.at[1 + t],
                send_sem=x_send.at[1 + t],
                recv_sem=x_recv.at[1 + t],
                device_id=(x_peer, my_y, my_z),
                device_id_type=MESH,
            )
            xr.start()
            x_rdmas.append(xr)

        bp = x_peer * half
        for i in range(N_Y):
            g = (my_y - i) % N_Y
            rows = bp + g * quarter
            rcv = pltpu.make_async_remote_copy(
                src_ref=quarter_buf,
                dst_ref=xblk.at[i],
                send_sem=x_send.at[i],
                recv_sem=x_recv.at[i],
                device_id=(x_peer, my_y, my_z),
                device_id_type=MESH,
            )
            rcv.wait_recv()
            cp = pltpu.make_async_copy(
                xblk.at[i], out_ref.at[pl.ds(rows, quarter), :],
                xout_sems.at[i],
            )
            cp.start()
            pending_copies.append(cp)
        for xr in x_rdmas:
            xr.wait_send()
        for cp in pending_copies:
            cp.wait()

    return pl.pallas_call(
        body,
        out_shape=jax.ShapeDtypeStruct((m, n), x.dtype),
        in_specs=[pl.BlockSpec(memory_space=pl.ANY)],
        out_specs=pl.BlockSpec(memory_space=pl.ANY),
        scratch_shapes=[
            pltpu.VMEM((N_Z - 1, piece, n), x.dtype),
            pltpu.VMEM((piece, n), x.dtype),
            pltpu.VMEM((quarter, n), x.dtype),
            pltpu.VMEM((N_Y - 1, quarter, n), x.dtype),
            pltpu.VMEM((N_Y, quarter, n), x.dtype),
            pltpu.SemaphoreType.DMA((N_Z - 1,)),
            pltpu.SemaphoreType.DMA((N_Z - 1,)),
            pltpu.SemaphoreType.DMA((N_Z - 1,)),
            pltpu.SemaphoreType.DMA((N_Z - 1,)),
            pltpu.SemaphoreType.DMA((N_Y - 1,)),
            pltpu.SemaphoreType.DMA((N_Y - 1,)),
            pltpu.SemaphoreType.DMA((N_Y,)),
            pltpu.SemaphoreType.DMA((N_Y,)),
            pltpu.SemaphoreType.DMA,
            pltpu.SemaphoreType.DMA((N_Y,)),
            pltpu.SemaphoreType.DMA((N_Y,)),
        ],
        compiler_params=pltpu.CompilerParams(
            collective_id=0, vmem_limit_bytes=48 * 1024 * 1024
        ),
    )(x)


# device time: 260121 ns/iter; 2.2844x vs baseline; 1.1454x over previous
import jax
import jax.numpy as jnp
from jax import lax
from jax.experimental import pallas as pl
from jax.experimental.pallas import tpu as pltpu

N_X, N_Y, N_Z = 2, 4, 4
MESH = pl.DeviceIdType.MESH


def kernel(x):
    m, n = x.shape
    half = m // N_X
    quarter = half // N_Y
    piece = quarter // N_Z

    def body(
        x_ref, out_ref,
        comm, x_stage, quarter_buf, yblk, xblk,
        zs_send, zs_recv, za_send, za_recv,
        y_send, y_recv, x_send, x_recv,
        copy_sem, out_sems, xout_sems,
    ):
        my_x = lax.axis_index("x")
        my_y = lax.axis_index("y")
        my_z = lax.axis_index("z")
        z_nxt = (my_z + 1) % N_Z
        z_prv = (my_z - 1) % N_Z
        y_nxt = (my_y + 1) % N_Y
        y_prv = (my_y - 1) % N_Y
        x_peer = 1 - my_x
        base_q = my_x * half + my_y * quarter

        pending_sends = []
        pending_copies = []

        def piece_id(i):
            return (my_z + 1 - i) % N_Z

        barrier_sem = pltpu.get_barrier_semaphore()
        for dev in (
            (my_x, my_y, z_nxt),
            (my_x, my_y, z_prv),
            (my_x, y_nxt, my_z),
            (my_x, y_prv, my_z),
            (x_peer, my_y, my_z),
        ):
            pl.semaphore_signal(
                barrier_sem, inc=1, device_id=dev, device_id_type=MESH
            )
        pl.semaphore_wait(barrier_sem, 5)

        def launch_piece(i):
            p = piece_id(i)
            sl = pl.ds(p * piece, piece)
            yr = pltpu.make_async_remote_copy(
                src_ref=quarter_buf.at[sl, :],
                dst_ref=yblk.at[0, sl, :],
                send_sem=y_send.at[i],
                recv_sem=y_recv.at[i],
                device_id=(my_x, y_nxt, my_z),
                device_id_type=MESH,
            )
            yr.start()
            pending_sends.append(yr)
            xr = pltpu.make_async_remote_copy(
                src_ref=quarter_buf.at[sl, :],
                dst_ref=xblk.at[0, sl, :],
                send_sem=x_send.at[i],
                recv_sem=x_recv.at[i],
                device_id=(x_peer, my_y, my_z),
                device_id_type=MESH,
            )
            xr.start()
            pending_sends.append(xr)

        for s in range(N_Z - 1):
            cs = (my_z - s) % N_Z
            cr = (my_z - s - 1) % N_Z
            src = (
                x_ref.at[pl.ds(base_q + cs * piece, piece), :]
                if s == 0
                else comm.at[s - 1]
            )
            rdma = pltpu.make_async_remote_copy(
                src_ref=src,
                dst_ref=comm.at[s],
                send_sem=zs_send.at[s],
                recv_sem=zs_recv.at[s],
                device_id=(my_x, my_y, z_nxt),
                device_id_type=MESH,
            )
            rdma.start()
            cp = pltpu.make_async_copy(
                x_ref.at[pl.ds(base_q + cr * piece, piece), :],
                x_stage,
                copy_sem,
            )
            cp.start()
            cp.wait()
            rdma.wait()
            comm[s] = comm[s] + x_stage[...]

        pstar = (my_z + 1) % N_Z
        cp = pltpu.make_async_copy(
            comm.at[N_Z - 2],
            quarter_buf.at[pl.ds(pstar * piece, piece), :],
            copy_sem,
        )
        cp.start()
        cp.wait()
        launch_piece(0)

        for s in range(N_Z - 1):
            pa = (my_z + 1 - s) % N_Z
            src = (
                comm.at[N_Z - 2]
                if s == 0
                else quarter_buf.at[pl.ds(pa * piece, piece), :]
            )
            rdma = pltpu.make_async_remote_copy(
                src_ref=src,
                dst_ref=quarter_buf.at[pl.ds(pa * piece, piece), :],
                send_sem=za_send.at[s],
                recv_sem=za_recv.at[s],
                device_id=(my_x, my_y, z_nxt),
                device_id_type=MESH,
            )
            rdma.start()
            rdma.wait()
            launch_piece(s + 1)

        cp = pltpu.make_async_copy(
            quarter_buf, out_ref.at[pl.ds(base_q, quarter), :], out_sems.at[0]
        )
        cp.start()
        pending_copies.append(cp)

        for t in range(N_Y - 1):
            g_r = (my_y - t - 1) % N_Y
            for i in range(N_Z):
                p = piece_id(i)
                sl = pl.ds(p * piece, piece)
                rcv = pltpu.make_async_remote_copy(
                    src_ref=quarter_buf.at[sl, :],
                    dst_ref=yblk.at[t, sl, :],
                    send_sem=y_send.at[t * N_Z + i],
                    recv_sem=y_recv.at[t * N_Z + i],
                    device_id=(my_x, y_nxt, my_z),
                    device_id_type=MESH,
                )
                rcv.wait_recv()
                if t < N_Y - 2:
                    yr = pltpu.make_async_remote_copy(
                        src_ref=yblk.at[t, sl, :],
                        dst_ref=yblk.at[t + 1, sl, :],
                        send_sem=y_send.at[(t + 1) * N_Z + i],
                        recv_sem=y_recv.at[(t + 1) * N_Z + i],
                        device_id=(my_x, y_nxt, my_z),
                        device_id_type=MESH,
                    )
                    yr.start()
                    pending_sends.append(yr)
                xr = pltpu.make_async_remote_copy(
                    src_ref=yblk.at[t, sl, :],
                    dst_ref=xblk.at[1 + t, sl, :],
                    send_sem=x_send.at[(1 + t) * N_Z + i],
                    recv_sem=x_recv.at[(1 + t) * N_Z + i],
                    device_id=(x_peer, my_y, my_z),
                    device_id_type=MESH,
                )
                xr.start()
                pending_sends.append(xr)
            rows = my_x * half + g_r * quarter
            cp = pltpu.make_async_copy(
                yblk.at[t], out_ref.at[pl.ds(rows, quarter), :],
                out_sems.at[1 + t],
            )
            cp.start()
            pending_copies.append(cp)

        bp = x_peer * half
        for b in range(N_Y):
            g = (my_y - b) % N_Y
            for i in range(N_Z):
                p = piece_id(i)
                sl = pl.ds(p * piece, piece)
                rcv = pltpu.make_async_remote_copy(
                    src_ref=quarter_buf.at[sl, :],
                    dst_ref=xblk.at[b, sl, :],
                    send_sem=x_send.at[b * N_Z + i],
                    recv_sem=x_recv.at[b * N_Z + i],
                    device_id=(x_peer, my_y, my_z),
                    device_id_type=MESH,
                )
                rcv.wait_recv()
            cp = pltpu.make_async_copy(
                xblk.at[b], out_ref.at[pl.ds(bp + g * quarter, quarter), :],
                xout_sems.at[b],
            )
            cp.start()
            pending_copies.append(cp)

        for r in pending_sends:
            r.wait_send()
        for cp in pending_copies:
            cp.wait()

    n_y_sems = (N_Y - 1) * N_Z
    n_x_sems = N_Y * N_Z

    return pl.pallas_call(
        body,
        out_shape=jax.ShapeDtypeStruct((m, n), x.dtype),
        in_specs=[pl.BlockSpec(memory_space=pl.ANY)],
        out_specs=pl.BlockSpec(memory_space=pl.ANY),
        scratch_shapes=[
            pltpu.VMEM((N_Z - 1, piece, n), x.dtype),
            pltpu.VMEM((piece, n), x.dtype),
            pltpu.VMEM((quarter, n), x.dtype),
            pltpu.VMEM((N_Y - 1, quarter, n), x.dtype),
            pltpu.VMEM((N_Y, quarter, n), x.dtype),
            pltpu.SemaphoreType.DMA((N_Z - 1,)),
            pltpu.SemaphoreType.DMA((N_Z - 1,)),
            pltpu.SemaphoreType.DMA((N_Z - 1,)),
            pltpu.SemaphoreType.DMA((N_Z - 1,)),
            pltpu.SemaphoreType.DMA((n_y_sems,)),
            pltpu.SemaphoreType.DMA((n_y_sems,)),
            pltpu.SemaphoreType.DMA((n_x_sems,)),
            pltpu.SemaphoreType.DMA((n_x_sems,)),
            pltpu.SemaphoreType.DMA,
            pltpu.SemaphoreType.DMA((N_Y,)),
            pltpu.SemaphoreType.DMA((N_Y,)),
        ],
        compiler_params=pltpu.CompilerParams(
            collective_id=0, vmem_limit_bytes=48 * 1024 * 1024
        ),
    )(x)


# device time: 251708 ns/iter; 2.3607x vs baseline; 1.0334x over previous
import jax
import jax.numpy as jnp
from jax import lax
from jax.experimental import pallas as pl
from jax.experimental.pallas import tpu as pltpu

N_X, N_Y, N_Z = 2, 4, 4
MESH = pl.DeviceIdType.MESH


def kernel(x):
    m, n = x.shape
    half = m // N_X
    quarter = half // N_Y
    piece = quarter // N_Z

    sub = piece // N_Z

    def body(
        x_ref, out_ref,
        comm, x_stage, quarter_buf, yblk, xblk,
        zs_send, zs_recv, za_send, za_recv,
        y_send, y_recv, x_send, x_recv,
        copy_sem, stage_sems, out_sems, xout_sems,
    ):
        my_x = lax.axis_index("x")
        my_y = lax.axis_index("y")
        my_z = lax.axis_index("z")
        z_nxt = (my_z + 1) % N_Z
        z_prv = (my_z - 1) % N_Z
        y_nxt = (my_y + 1) % N_Y
        y_prv = (my_y - 1) % N_Y
        x_peer = 1 - my_x
        base_q = my_x * half + my_y * quarter

        pending_sends = []
        pending_copies = []

        def piece_id(i):
            return (my_z + 1 - i) % N_Z

        barrier_sem = pltpu.get_barrier_semaphore()
        for dev in (
            (my_x, my_y, z_nxt),
            (my_x, my_y, z_prv),
            (my_x, y_nxt, my_z),
            (my_x, y_prv, my_z),
            (x_peer, my_y, my_z),
        ):
            pl.semaphore_signal(
                barrier_sem, inc=1, device_id=dev, device_id_type=MESH
            )
        pl.semaphore_wait(barrier_sem, 5)

        def launch_piece(i):
            p = piece_id(i)
            sl = pl.ds(p * piece, piece)
            yr = pltpu.make_async_remote_copy(
                src_ref=quarter_buf.at[sl, :],
                dst_ref=yblk.at[0, sl, :],
                send_sem=y_send.at[i],
                recv_sem=y_recv.at[i],
                device_id=(my_x, y_nxt, my_z),
                device_id_type=MESH,
            )
            yr.start()
            pending_sends.append(yr)
            xr = pltpu.make_async_remote_copy(
                src_ref=quarter_buf.at[sl, :],
                dst_ref=xblk.at[0, sl, :],
                send_sem=x_send.at[i],
                recv_sem=x_recv.at[i],
                device_id=(x_peer, my_y, my_z),
                device_id_type=MESH,
            )
            xr.start()
            pending_sends.append(xr)

        pre = []
        for s in range(N_Z - 1):
            cr = (my_z - s - 1) % N_Z
            cp = pltpu.make_async_copy(
                x_ref.at[pl.ds(base_q + cr * piece, piece), :],
                x_stage.at[s],
                stage_sems.at[s],
            )
            cp.start()
            pre.append(cp)

        def rs_rdma(s, j):
            cs = (my_z - s) % N_Z
            off = j * sub
            src = (
                x_ref.at[pl.ds(base_q + cs * piece + off, sub), :]
                if s == 0
                else comm.at[s - 1, pl.ds(off, sub), :]
            )
            return pltpu.make_async_remote_copy(
                src_ref=src,
                dst_ref=comm.at[s, pl.ds(off, sub), :],
                send_sem=zs_send.at[s * N_Z + j],
                recv_sem=zs_recv.at[s * N_Z + j],
                device_id=(my_x, my_y, z_nxt),
                device_id_type=MESH,
            )

        def ag_rdma(s, j):
            pa = (my_z + 1 - s) % N_Z
            off = j * sub
            src = (
                comm.at[N_Z - 2, pl.ds(off, sub), :]
                if s == 0
                else quarter_buf.at[pl.ds(pa * piece + off, sub), :]
            )
            return pltpu.make_async_remote_copy(
                src_ref=src,
                dst_ref=quarter_buf.at[pl.ds(pa * piece + off, sub), :],
                send_sem=za_send.at[s * N_Z + j],
                recv_sem=za_recv.at[s * N_Z + j],
                device_id=(my_x, my_y, z_nxt),
                device_id_type=MESH,
            )

        rs_descs = {}
        ag_descs = {}
        for j in range(N_Z):
            d = rs_rdma(0, j)
            d.start()
            rs_descs[(0, j)] = d
        for s in range(N_Z - 1):
            pre[s].wait()
            for j in range(N_Z):
                rs_descs[(s, j)].wait()
                off = pl.ds(j * sub, sub)
                comm[s, off, :] = comm[s, off, :] + x_stage[s, off, :]
                if s < N_Z - 2:
                    d = rs_rdma(s + 1, j)
                    d.start()
                    rs_descs[(s + 1, j)] = d
                else:
                    d = ag_rdma(0, j)
                    d.start()
                    ag_descs[(0, j)] = d

        pstar = (my_z + 1) % N_Z
        cp = pltpu.make_async_copy(
            comm.at[N_Z - 2],
            quarter_buf.at[pl.ds(pstar * piece, piece), :],
            copy_sem,
        )
        cp.start()
        cp.wait()
        launch_piece(0)

        for s in range(N_Z - 1):
            for j in range(N_Z):
                ag_descs[(s, j)].wait()
                if s < N_Z - 2:
                    d = ag_rdma(s + 1, j)
                    d.start()
                    ag_descs[(s + 1, j)] = d
            launch_piece(s + 1)

        cp = pltpu.make_async_copy(
            quarter_buf, out_ref.at[pl.ds(base_q, quarter), :], out_sems.at[0]
        )
        cp.start()
        pending_copies.append(cp)

        for t in range(N_Y - 1):
            g_r = (my_y - t - 1) % N_Y
            for i in range(N_Z):
                p = piece_id(i)
                sl = pl.ds(p * piece, piece)
                rcv = pltpu.make_async_remote_copy(
                    src_ref=quarter_buf.at[sl, :],
                    dst_ref=yblk.at[t, sl, :],
                    send_sem=y_send.at[t * N_Z + i],
                    recv_sem=y_recv.at[t * N_Z + i],
                    device_id=(my_x, y_nxt, my_z),
                    device_id_type=MESH,
                )
                rcv.wait_recv()
                if t < N_Y - 2:
                    yr = pltpu.make_async_remote_copy(
                        src_ref=yblk.at[t, sl, :],
                        dst_ref=yblk.at[t + 1, sl, :],
                        send_sem=y_send.at[(t + 1) * N_Z + i],
                        recv_sem=y_recv.at[(t + 1) * N_Z + i],
                        device_id=(my_x, y_nxt, my_z),
                        device_id_type=MESH,
                    )
                    yr.start()
                    pending_sends.append(yr)
                xr = pltpu.make_async_remote_copy(
                    src_ref=yblk.at[t, sl, :],
                    dst_ref=xblk.at[1 + t, sl, :],
                    send_sem=x_send.at[(1 + t) * N_Z + i],
                    recv_sem=x_recv.at[(1 + t) * N_Z + i],
                    device_id=(x_peer, my_y, my_z),
                    device_id_type=MESH,
                )
                xr.start()
                pending_sends.append(xr)
            rows = my_x * half + g_r * quarter
            cp = pltpu.make_async_copy(
                yblk.at[t], out_ref.at[pl.ds(rows, quarter), :],
                out_sems.at[1 + t],
            )
            cp.start()
            pending_copies.append(cp)

        bp = x_peer * half
        for b in range(N_Y):
            g = (my_y - b) % N_Y
            for i in range(N_Z):
                p = piece_id(i)
                sl = pl.ds(p * piece, piece)
                rcv = pltpu.make_async_remote_copy(
                    src_ref=quarter_buf.at[sl, :],
                    dst_ref=xblk.at[b, sl, :],
                    send_sem=x_send.at[b * N_Z + i],
                    recv_sem=x_recv.at[b * N_Z + i],
                    device_id=(x_peer, my_y, my_z),
                    device_id_type=MESH,
                )
                rcv.wait_recv()
            cp = pltpu.make_async_copy(
                xblk.at[b], out_ref.at[pl.ds(bp + g * quarter, quarter), :],
                xout_sems.at[b],
            )
            cp.start()
            pending_copies.append(cp)

        for r in pending_sends:
            r.wait_send()
        for cp in pending_copies:
            cp.wait()

    n_y_sems = (N_Y - 1) * N_Z
    n_x_sems = N_Y * N_Z

    return pl.pallas_call(
        body,
        out_shape=jax.ShapeDtypeStruct((m, n), x.dtype),
        in_specs=[pl.BlockSpec(memory_space=pl.ANY)],
        out_specs=pl.BlockSpec(memory_space=pl.ANY),
        scratch_shapes=[
            pltpu.VMEM((N_Z - 1, piece, n), x.dtype),
            pltpu.VMEM((N_Z - 1, piece, n), x.dtype),
            pltpu.VMEM((quarter, n), x.dtype),
            pltpu.VMEM((N_Y - 1, quarter, n), x.dtype),
            pltpu.VMEM((N_Y, quarter, n), x.dtype),
            pltpu.SemaphoreType.DMA(((N_Z - 1) * N_Z,)),
            pltpu.SemaphoreType.DMA(((N_Z - 1) * N_Z,)),
            pltpu.SemaphoreType.DMA(((N_Z - 1) * N_Z,)),
            pltpu.SemaphoreType.DMA(((N_Z - 1) * N_Z,)),
            pltpu.SemaphoreType.DMA((n_y_sems,)),
            pltpu.SemaphoreType.DMA((n_y_sems,)),
            pltpu.SemaphoreType.DMA((n_x_sems,)),
            pltpu.SemaphoreType.DMA((n_x_sems,)),
            pltpu.SemaphoreType.DMA,
            pltpu.SemaphoreType.DMA((N_Z - 1,)),
            pltpu.SemaphoreType.DMA((N_Y,)),
            pltpu.SemaphoreType.DMA((N_Y,)),
        ],
        compiler_params=pltpu.CompilerParams(
            collective_id=0, vmem_limit_bytes=48 * 1024 * 1024
        ),
    )(x)
